# baseline (speedup 1.0000x reference)
import functools

import jax
import jax.numpy as jnp
from jax import lax
from jax.experimental import pallas as pl
from jax.experimental.pallas import tpu as pltpu
from jax.experimental.pallas import tpu_sc as plsc

BATCH = 16384
EMB_K = 16
NUM_ROWS = 1000000
NUM_WORKERS = 32
ROWS_PER_WORKER = BATCH // NUM_WORKERS
ELEMS_PER_WORKER = ROWS_PER_WORKER * EMB_K

BLK = 256
NUM_BLOCKS = NUM_ROWS // BLK
FULL_BLOCKS = NUM_ROWS // BLK
TAIL_ROWS = NUM_ROWS - FULL_BLOCKS * BLK


def _compact_one(tbl_hbm, out_flat, buf0, buf1, flat_v, sem0, sem1, wid):
    def fire(b, buf, sem):
        lo = (b * NUM_WORKERS + wid) * BLK
        pltpu.make_async_copy(tbl_hbm.at[pl.ds(lo, BLK)], buf, sem).start()

    def compact(b, buf, sem):
        lo = (b * NUM_WORKERS + wid) * BLK
        pltpu.make_async_copy(tbl_hbm.at[pl.ds(0, BLK)], buf, sem).wait()
        for r in range(BLK):
            flat_v[pl.ds(r * EMB_K, EMB_K)] = buf[r]
        pltpu.sync_copy(flat_v, out_flat.at[pl.ds(lo * EMB_K,
                                                  BLK * EMB_K)])

    nb = 122 + jnp.where(wid < 2, 1, 0)

    fire(0, buf0, sem0)

    def pair_body(t, _):
        b0 = 2 * t
        b1 = 2 * t + 1

        @pl.when(b1 < nb)
        def _():
            fire(b1, buf1, sem1)

        compact(b0, buf0, sem0)

        @pl.when(b1 < nb)
        def _():
            @pl.when(b1 + 1 < nb)
            def _():
                fire(b1 + 1, buf0, sem0)

            compact(b1, buf1, sem1)

        return ()

    lax.fori_loop(0, (nb + 1) // 2, pair_body, (), unroll=False)

    @pl.when(wid == NUM_WORKERS - 1)
    def _():
        lo = FULL_BLOCKS * BLK
        pltpu.sync_copy(tbl_hbm.at[pl.ds(lo, TAIL_ROWS)],
                        buf0.at[pl.ds(0, TAIL_ROWS)])
        for r in range(TAIL_ROWS):
            flat_v[pl.ds(r * EMB_K, EMB_K)] = buf0[r]
        pltpu.sync_copy(flat_v.at[pl.ds(0, TAIL_ROWS * EMB_K)],
                        out_flat.at[pl.ds(lo * EMB_K, TAIL_ROWS * EMB_K)])


def _compact_body(w_hbm, h_hbm, wf_out, hf_out,
                  buf0, buf1, flat_v, sem0, sem1):
    wid = lax.axis_index("s") * 2 + lax.axis_index("c")
    _compact_one(w_hbm, wf_out, buf0, buf1, flat_v, sem0, sem1, wid)
    _compact_one(h_hbm, hf_out, buf0, buf1, flat_v, sem0, sem1, wid)


_compact_call = functools.partial(
    pl.kernel,
    out_type=(
        jax.ShapeDtypeStruct((NUM_ROWS * EMB_K,), jnp.float32),
        jax.ShapeDtypeStruct((NUM_ROWS * EMB_K,), jnp.float32),
    ),
    mesh=plsc.VectorSubcoreMesh(core_axis_name="c", subcore_axis_name="s"),
    scratch_types=[
        pltpu.VMEM((BLK, EMB_K), jnp.float32),
        pltpu.VMEM((BLK, EMB_K), jnp.float32),
        pltpu.VMEM((BLK * EMB_K,), jnp.float32),
        pltpu.SemaphoreType.DMA,
        pltpu.SemaphoreType.DMA,
    ],
)(_compact_body)


def _lookup_body(ufidx_hbm, vfidx_hbm, wf_hbm, hf_hbm, u_out, v_out,
                 uf_v, vf_v, ue_v, ve_v, sem_u, sem_v):
    wid = lax.axis_index("s") * 2 + lax.axis_index("c")
    base = wid * ELEMS_PER_WORKER
    pltpu.sync_copy(ufidx_hbm.at[pl.ds(base, ELEMS_PER_WORKER)], uf_v)
    pltpu.sync_copy(vfidx_hbm.at[pl.ds(base, ELEMS_PER_WORKER)], vf_v)
    cp_u = pltpu.make_async_copy(wf_hbm.at[uf_v], ue_v, sem_u)
    cp_v = pltpu.make_async_copy(hf_hbm.at[vf_v], ve_v, sem_v)
    cp_u.start()
    cp_v.start()
    cp_u.wait()
    cp_v.wait()
    pltpu.sync_copy(ue_v, u_out.at[pl.ds(base, ELEMS_PER_WORKER)])
    pltpu.sync_copy(ve_v, v_out.at[pl.ds(base, ELEMS_PER_WORKER)])


_lookup_call = functools.partial(
    pl.kernel,
    out_type=(
        jax.ShapeDtypeStruct((BATCH * EMB_K,), jnp.float32),
        jax.ShapeDtypeStruct((BATCH * EMB_K,), jnp.float32),
    ),
    mesh=plsc.VectorSubcoreMesh(core_axis_name="c", subcore_axis_name="s"),
    compiler_params=pltpu.CompilerParams(use_tc_tiling_on_sc=False),
    scratch_types=[
        pltpu.VMEM((ELEMS_PER_WORKER,), jnp.int32),
        pltpu.VMEM((ELEMS_PER_WORKER,), jnp.int32),
        pltpu.VMEM((ELEMS_PER_WORKER,), jnp.float32),
        pltpu.VMEM((ELEMS_PER_WORKER,), jnp.float32),
        pltpu.SemaphoreType.DMA,
        pltpu.SemaphoreType.DMA,
    ],
)(_lookup_body)


def _mlp_body(u_ref, v_ref, w1_ref, b1_ref, w2_ref, o_ref):
    u = u_ref[...]
    v = v_ref[...]
    w1a = w1_ref[0:EMB_K, :]
    w1b = w1_ref[EMB_K:2 * EMB_K, :]
    h = jnp.dot(u, w1a, preferred_element_type=jnp.float32)
    h = h + jnp.dot(v, w1b, preferred_element_type=jnp.float32)
    h = jnp.maximum(h + b1_ref[...], 0.0)
    o_ref[...] = jnp.sum(h * w2_ref[...], axis=1, keepdims=True)


def _mlp_call(u, v, w1, b1_row, w2_row):
    return pl.pallas_call(
        _mlp_body,
        out_shape=jax.ShapeDtypeStruct((BATCH, 1), jnp.float32),
    )(u, v, w1, b1_row, w2_row)


def kernel(x, W, H, W1, b1, W2):
    lane = jnp.arange(EMB_K, dtype=jnp.int32)
    ufidx = (x[:, 0].astype(jnp.int32)[:, None] * EMB_K + lane).reshape(-1)
    vfidx = (x[:, 1].astype(jnp.int32)[:, None] * EMB_K + lane).reshape(-1)
    wf, hf = _compact_call(W, H)
    u_flat, v_flat = _lookup_call(ufidx, vfidx, wf, hf)
    u_rows = u_flat.reshape(BATCH, EMB_K)
    v_rows = v_flat.reshape(BATCH, EMB_K)
    return _mlp_call(u_rows, v_rows, W1, b1.reshape(1, EMB_K),
                     W2.reshape(1, EMB_K))


# speedup vs baseline: 1.6251x; 1.6251x over previous
import functools

import jax
import jax.numpy as jnp
from jax import lax
from jax.experimental import pallas as pl
from jax.experimental.pallas import tpu as pltpu
from jax.experimental.pallas import tpu_sc as plsc

BATCH = 16384
EMB_K = 16
SC_ROWS = 8192
TC_ROWS = BATCH - SC_ROWS
NUM_WORKERS = 32
ROWS_PER_WORKER = SC_ROWS // NUM_WORKERS
GROUP = 16
LAG = 4
CHUNK = 256
NUM_PASSES = ROWS_PER_WORKER // CHUNK
GROUPS_PER_PASS = CHUNK // GROUP
TGROUP = 16
TLAG = 8
TGROUPS = TC_ROWS // TGROUP


def _gather_body(uidx_hbm, vidx_hbm, w_hbm, h_hbm, u_out, v_out,
                 uidx_v, vidx_v, u_v, v_v, sem_u, sem_v):
    wid = lax.axis_index("s") * 2 + lax.axis_index("c")
    base = wid * ROWS_PER_WORKER
    pltpu.sync_copy(uidx_hbm.at[pl.ds(base, ROWS_PER_WORKER)], uidx_v)
    pltpu.sync_copy(vidx_hbm.at[pl.ds(base, ROWS_PER_WORKER)], vidx_v)

    for p in range(NUM_PASSES):
        def fire(g, p=p):
            uvec = uidx_v[pl.ds(p * CHUNK + g * GROUP, GROUP)]
            vvec = vidx_v[pl.ds(p * CHUNK + g * GROUP, GROUP)]
            for j in range(GROUP):
                i = g * GROUP + j
                ru = uvec[j]
                rv = vvec[j]
                pltpu.make_async_copy(
                    w_hbm.at[pl.ds(ru, 1)], u_v.at[pl.ds(i, 1)],
                    sem_u).start()
                pltpu.make_async_copy(
                    h_hbm.at[pl.ds(rv, 1)], v_v.at[pl.ds(i, 1)],
                    sem_v).start()

        def drain(g):
            pltpu.make_async_copy(
                w_hbm.at[pl.ds(0, GROUP)],
                u_v.at[pl.ds(g * GROUP, GROUP)], sem_u).wait()
            pltpu.make_async_copy(
                h_hbm.at[pl.ds(0, GROUP)],
                v_v.at[pl.ds(g * GROUP, GROUP)], sem_v).wait()

        for g0 in range(LAG):
            fire(g0)

        def loop_body(g, _):
            fire_g = g + LAG

            @pl.when(fire_g < GROUPS_PER_PASS)
            def _():
                fire(fire_g)

            drain(g)
            return ()

        lax.fori_loop(0, GROUPS_PER_PASS, loop_body, (), unroll=False)

        pltpu.sync_copy(u_v, u_out.at[pl.ds(base + p * CHUNK, CHUNK)])
        pltpu.sync_copy(v_v, v_out.at[pl.ds(base + p * CHUNK, CHUNK)])


_gather_call = functools.partial(
    pl.kernel,
    out_type=(
        jax.ShapeDtypeStruct((SC_ROWS, EMB_K), jnp.float32),
        jax.ShapeDtypeStruct((SC_ROWS, EMB_K), jnp.float32),
    ),
    mesh=plsc.VectorSubcoreMesh(core_axis_name="c", subcore_axis_name="s"),
    scratch_types=[
        pltpu.VMEM((ROWS_PER_WORKER,), jnp.int32),
        pltpu.VMEM((ROWS_PER_WORKER,), jnp.int32),
        pltpu.VMEM((CHUNK, EMB_K), jnp.float32),
        pltpu.VMEM((CHUNK, EMB_K), jnp.float32),
        pltpu.SemaphoreType.DMA,
        pltpu.SemaphoreType.DMA,
    ],
)(_gather_body)


def _tc_gather_body(uidx_sref, vidx_sref, w_ref, h_ref, u_out, v_out,
                    sem_u, sem_v):
    def fire(g):
        for j in range(TGROUP):
            i = g * TGROUP + j
            ru = uidx_sref[i]
            rv = vidx_sref[i]
            pltpu.make_async_copy(
                w_ref.at[pl.ds(ru, 1)], u_out.at[pl.ds(i, 1)],
                sem_u).start()
            pltpu.make_async_copy(
                h_ref.at[pl.ds(rv, 1)], v_out.at[pl.ds(i, 1)],
                sem_v).start()

    def drain(g):
        pltpu.make_async_copy(
            w_ref.at[pl.ds(0, TGROUP)],
            u_out.at[pl.ds(g * TGROUP, TGROUP)], sem_u).wait()
        pltpu.make_async_copy(
            h_ref.at[pl.ds(0, TGROUP)],
            v_out.at[pl.ds(g * TGROUP, TGROUP)], sem_v).wait()

    for g0 in range(TLAG):
        fire(g0)

    def loop_body(g, _):
        fire_g = g + TLAG

        @pl.when(fire_g < TGROUPS)
        def _():
            fire(fire_g)

        drain(g)
        return ()

    lax.fori_loop(0, TGROUPS, loop_body, (), unroll=False)


def _tc_gather_call(uidx_tc, vidx_tc, w, h):
    return pl.pallas_call(
        _tc_gather_body,
        in_specs=[
            pl.BlockSpec(memory_space=pltpu.SMEM),
            pl.BlockSpec(memory_space=pltpu.SMEM),
            pl.BlockSpec(memory_space=pl.ANY),
            pl.BlockSpec(memory_space=pl.ANY),
        ],
        out_shape=(
            jax.ShapeDtypeStruct((TC_ROWS, EMB_K), jnp.float32),
            jax.ShapeDtypeStruct((TC_ROWS, EMB_K), jnp.float32),
        ),
        scratch_shapes=[pltpu.SemaphoreType.DMA, pltpu.SemaphoreType.DMA],
    )(uidx_tc, vidx_tc, w, h)


def _mlp_body(u_ref, v_ref, w1_ref, b1_ref, w2_ref, o_ref):
    u = u_ref[...]
    v = v_ref[...]
    w1a = w1_ref[0:EMB_K, :]
    w1b = w1_ref[EMB_K:2 * EMB_K, :]
    h = jnp.dot(u, w1a, preferred_element_type=jnp.float32)
    h = h + jnp.dot(v, w1b, preferred_element_type=jnp.float32)
    h = jnp.maximum(h + b1_ref[...], 0.0)
    o_ref[...] = jnp.sum(h * w2_ref[...], axis=1, keepdims=True)


def _mlp_call(u, v, w1, b1_row, w2_row):
    return pl.pallas_call(
        _mlp_body,
        out_shape=jax.ShapeDtypeStruct((BATCH, 1), jnp.float32),
    )(u, v, w1, b1_row, w2_row)


def kernel(x, W, H, W1, b1, W2):
    uidx = x[:, 0].astype(jnp.int32)
    vidx = x[:, 1].astype(jnp.int32)
    u_sc, v_sc = _gather_call(uidx[:SC_ROWS], vidx[:SC_ROWS], W, H)
    u_tc, v_tc = _tc_gather_call(uidx[SC_ROWS:], vidx[SC_ROWS:], W, H)
    u_rows = jnp.concatenate([u_sc, u_tc], axis=0)
    v_rows = jnp.concatenate([v_sc, v_tc], axis=0)
    return _mlp_call(u_rows, v_rows, W1, b1.reshape(1, EMB_K),
                     W2.reshape(1, EMB_K))


# speedup vs baseline: 1.7773x; 1.0936x over previous
import functools

import jax
import jax.numpy as jnp
from jax import lax
from jax.experimental import pallas as pl
from jax.experimental.pallas import tpu as pltpu
from jax.experimental.pallas import tpu_sc as plsc

BATCH = 16384
EMB_K = 16
SC_ROWS = 8192
TC_ROWS = BATCH - SC_ROWS
NUM_WORKERS = 32
ROWS_PER_WORKER = SC_ROWS // NUM_WORKERS
GROUP = 16
LAG = 4
CHUNK = 256
NUM_PASSES = ROWS_PER_WORKER // CHUNK
GROUPS_PER_PASS = CHUNK // GROUP
TGROUP = 16
TLAG = 8
TGROUPS = TC_ROWS // TGROUP


def _gather_body(uidx_hbm, vidx_hbm, w_hbm, h_hbm, u_out, v_out,
                 uidx_v, vidx_v, u_v, v_v, sem_u, sem_v):
    wid = lax.axis_index("s") * 2 + lax.axis_index("c")
    base = wid * ROWS_PER_WORKER
    pltpu.sync_copy(uidx_hbm.at[pl.ds(base, ROWS_PER_WORKER)], uidx_v)
    pltpu.sync_copy(vidx_hbm.at[pl.ds(base, ROWS_PER_WORKER)], vidx_v)

    for p in range(NUM_PASSES):
        def fire(g, p=p):
            uvec = uidx_v[pl.ds(p * CHUNK + g * GROUP, GROUP)]
            vvec = vidx_v[pl.ds(p * CHUNK + g * GROUP, GROUP)]
            for j in range(GROUP):
                i = g * GROUP + j
                ru = uvec[j]
                rv = vvec[j]
                pltpu.make_async_copy(
                    w_hbm.at[pl.ds(ru, 1)], u_v.at[pl.ds(i, 1)],
                    sem_u).start()
                pltpu.make_async_copy(
                    h_hbm.at[pl.ds(rv, 1)], v_v.at[pl.ds(i, 1)],
                    sem_v).start()

        def drain(g):
            pltpu.make_async_copy(
                w_hbm.at[pl.ds(0, GROUP)],
                u_v.at[pl.ds(g * GROUP, GROUP)], sem_u).wait()
            pltpu.make_async_copy(
                h_hbm.at[pl.ds(0, GROUP)],
                v_v.at[pl.ds(g * GROUP, GROUP)], sem_v).wait()

        for g0 in range(LAG):
            fire(g0)

        def loop_body(g, _):
            fire_g = g + LAG

            @pl.when(fire_g < GROUPS_PER_PASS)
            def _():
                fire(fire_g)

            drain(g)
            return ()

        lax.fori_loop(0, GROUPS_PER_PASS, loop_body, (), unroll=False)

        pltpu.sync_copy(u_v, u_out.at[pl.ds(base + p * CHUNK, CHUNK)])
        pltpu.sync_copy(v_v, v_out.at[pl.ds(base + p * CHUNK, CHUNK)])


_gather_call = functools.partial(
    pl.kernel,
    out_type=(
        jax.ShapeDtypeStruct((SC_ROWS, EMB_K), jnp.float32),
        jax.ShapeDtypeStruct((SC_ROWS, EMB_K), jnp.float32),
    ),
    mesh=plsc.VectorSubcoreMesh(core_axis_name="c", subcore_axis_name="s"),
    scratch_types=[
        pltpu.VMEM((ROWS_PER_WORKER,), jnp.int32),
        pltpu.VMEM((ROWS_PER_WORKER,), jnp.int32),
        pltpu.VMEM((CHUNK, EMB_K), jnp.float32),
        pltpu.VMEM((CHUNK, EMB_K), jnp.float32),
        pltpu.SemaphoreType.DMA,
        pltpu.SemaphoreType.DMA,
    ],
)(_gather_body)


def _tc_gather_body(uidx_sref, vidx_sref, w_ref, h_ref, u_out, v_out,
                    sem_u, sem_v):
    def fire(g):
        for j in range(TGROUP):
            i = g * TGROUP + j
            ru = uidx_sref[i]
            rv = vidx_sref[i]
            pltpu.make_async_copy(
                w_ref.at[pl.ds(ru, 1)], u_out.at[pl.ds(i, 1)],
                sem_u).start()
            pltpu.make_async_copy(
                h_ref.at[pl.ds(rv, 1)], v_out.at[pl.ds(i, 1)],
                sem_v).start()

    def drain(g):
        pltpu.make_async_copy(
            w_ref.at[pl.ds(0, TGROUP)],
            u_out.at[pl.ds(g * TGROUP, TGROUP)], sem_u).wait()
        pltpu.make_async_copy(
            h_ref.at[pl.ds(0, TGROUP)],
            v_out.at[pl.ds(g * TGROUP, TGROUP)], sem_v).wait()

    for g0 in range(TLAG):
        fire(g0)

    def loop_body(g, _):
        fire_g = g + TLAG

        @pl.when(fire_g < TGROUPS)
        def _():
            fire(fire_g)

        drain(g)
        return ()

    lax.fori_loop(0, TGROUPS, loop_body, (), unroll=False)


def _tc_gather_call(uidx_tc, vidx_tc, w, h):
    return pl.pallas_call(
        _tc_gather_body,
        in_specs=[
            pl.BlockSpec(memory_space=pltpu.SMEM),
            pl.BlockSpec(memory_space=pltpu.SMEM),
            pl.BlockSpec(memory_space=pl.ANY),
            pl.BlockSpec(memory_space=pl.ANY),
        ],
        out_shape=(
            jax.ShapeDtypeStruct((TC_ROWS, EMB_K), jnp.float32),
            jax.ShapeDtypeStruct((TC_ROWS, EMB_K), jnp.float32),
        ),
        scratch_shapes=[pltpu.SemaphoreType.DMA, pltpu.SemaphoreType.DMA],
    )(uidx_tc, vidx_tc, w, h)


def _mlp_body(u_ref, v_ref, w1_ref, b1_ref, w2_ref, o_ref):
    u = u_ref[...]
    v = v_ref[...]
    w1a = w1_ref[0:EMB_K, :]
    w1b = w1_ref[EMB_K:2 * EMB_K, :]
    h = jnp.dot(u, w1a, preferred_element_type=jnp.float32)
    h = h + jnp.dot(v, w1b, preferred_element_type=jnp.float32)
    h = jnp.maximum(h + b1_ref[...], 0.0)
    o_ref[...] = jnp.sum(h * w2_ref[...], axis=1, keepdims=True)


def _mlp_call(u, v, w1, b1_row, w2_row):
    return pl.pallas_call(
        _mlp_body,
        out_shape=jax.ShapeDtypeStruct((BATCH, 1), jnp.float32),
    )(u, v, w1, b1_row, w2_row)


def kernel(x, W, H, W1, b1, W2):
    uidx = x[:, 0].astype(jnp.int32)
    vidx = x[:, 1].astype(jnp.int32)
    u_sc, v_sc = _gather_call(uidx[:SC_ROWS], vidx[:SC_ROWS], W, H)
    u_sc2, v_sc2 = _gather_call(uidx[SC_ROWS:], vidx[SC_ROWS:], W, H)
    u_rows = jnp.concatenate([u_sc, u_sc2], axis=0)
    v_rows = jnp.concatenate([v_sc, v_sc2], axis=0)
    return _mlp_call(u_rows, v_rows, W1, b1.reshape(1, EMB_K),
                     W2.reshape(1, EMB_K))


# speedup vs baseline: 1.8467x; 1.0390x over previous
import functools

import jax
import jax.numpy as jnp
from jax import lax
from jax.experimental import pallas as pl
from jax.experimental.pallas import tpu as pltpu
from jax.experimental.pallas import tpu_sc as plsc

BATCH = 16384
EMB_K = 16
NUM_WORKERS = 32
ROWS_PER_WORKER = BATCH // NUM_WORKERS
GROUP = 16
LAG = 4
CHUNK = 256
NUM_PASSES = ROWS_PER_WORKER // CHUNK
GROUPS_PER_PASS = CHUNK // GROUP


def _gather_body(uidx_hbm, vidx_hbm, w_hbm, h_hbm, u_out, v_out,
                 uidx_v, vidx_v, u_v, v_v, sem_u, sem_v):
    wid = lax.axis_index("s") * 2 + lax.axis_index("c")
    base = wid * ROWS_PER_WORKER
    pltpu.sync_copy(uidx_hbm.at[pl.ds(base, ROWS_PER_WORKER)], uidx_v)
    pltpu.sync_copy(vidx_hbm.at[pl.ds(base, ROWS_PER_WORKER)], vidx_v)

    for p in range(NUM_PASSES):
        def fire(g, p=p):
            uvec = uidx_v[pl.ds(p * CHUNK + g * GROUP, GROUP)]
            vvec = vidx_v[pl.ds(p * CHUNK + g * GROUP, GROUP)]
            for j in range(GROUP):
                i = g * GROUP + j
                ru = uvec[j]
                rv = vvec[j]
                pltpu.make_async_copy(
                    w_hbm.at[pl.ds(ru, 1)], u_v.at[pl.ds(i, 1)],
                    sem_u).start()
                pltpu.make_async_copy(
                    h_hbm.at[pl.ds(rv, 1)], v_v.at[pl.ds(i, 1)],
                    sem_v).start()

        def drain(g):
            pltpu.make_async_copy(
                w_hbm.at[pl.ds(0, GROUP)],
                u_v.at[pl.ds(g * GROUP, GROUP)], sem_u).wait()
            pltpu.make_async_copy(
                h_hbm.at[pl.ds(0, GROUP)],
                v_v.at[pl.ds(g * GROUP, GROUP)], sem_v).wait()

        for g0 in range(LAG):
            fire(g0)

        def loop_body(g, _):
            fire_g = g + LAG

            @pl.when(fire_g < GROUPS_PER_PASS)
            def _():
                fire(fire_g)

            drain(g)
            return ()

        lax.fori_loop(0, GROUPS_PER_PASS, loop_body, (), unroll=False)

        pltpu.sync_copy(u_v, u_out.at[pl.ds(base + p * CHUNK, CHUNK)])
        pltpu.sync_copy(v_v, v_out.at[pl.ds(base + p * CHUNK, CHUNK)])


_gather_call = functools.partial(
    pl.kernel,
    out_type=(
        jax.ShapeDtypeStruct((BATCH, EMB_K), jnp.float32),
        jax.ShapeDtypeStruct((BATCH, EMB_K), jnp.float32),
    ),
    mesh=plsc.VectorSubcoreMesh(core_axis_name="c", subcore_axis_name="s"),
    compiler_params=pltpu.CompilerParams(use_tc_tiling_on_sc=True),
    scratch_types=[
        pltpu.VMEM((ROWS_PER_WORKER,), jnp.int32),
        pltpu.VMEM((ROWS_PER_WORKER,), jnp.int32),
        pltpu.VMEM((CHUNK, EMB_K), jnp.float32),
        pltpu.VMEM((CHUNK, EMB_K), jnp.float32),
        pltpu.SemaphoreType.DMA,
        pltpu.SemaphoreType.DMA,
    ],
)(_gather_body)


def _mlp_body(u_ref, v_ref, w1_ref, b1_ref, w2_ref, o_ref):
    u = u_ref[...]
    v = v_ref[...]
    w1a = w1_ref[0:EMB_K, :]
    w1b = w1_ref[EMB_K:2 * EMB_K, :]
    h = jnp.dot(u, w1a, preferred_element_type=jnp.float32)
    h = h + jnp.dot(v, w1b, preferred_element_type=jnp.float32)
    h = jnp.maximum(h + b1_ref[...], 0.0)
    o_ref[...] = jnp.sum(h * w2_ref[...], axis=1, keepdims=True)


def _mlp_call(u, v, w1, b1_row, w2_row):
    return pl.pallas_call(
        _mlp_body,
        out_shape=jax.ShapeDtypeStruct((BATCH, 1), jnp.float32),
    )(u, v, w1, b1_row, w2_row)


def kernel(x, W, H, W1, b1, W2):
    uidx = x[:, 0].astype(jnp.int32)
    vidx = x[:, 1].astype(jnp.int32)
    u_rows, v_rows = _gather_call(uidx, vidx, W, H)
    return _mlp_call(u_rows, v_rows, W1, b1.reshape(1, EMB_K),
                     W2.reshape(1, EMB_K))
